# Initial kernel scaffold; baseline (speedup 1.0000x reference)
import jax
import jax.numpy as jnp
from jax.experimental import pallas as pl


def kernel(x, W1, b1, nodes, edge_dst, edge_src):
    raise NotImplementedError("write your pallas kernel here")


# speedup vs baseline: 3.5817x; 3.5817x over previous
import functools

import jax
import jax.numpy as jnp
from jax import lax
from jax.experimental import pallas as pl
from jax.experimental.pallas import tpu as pltpu
from jax.experimental.pallas import tpu_sc as plsc

A_THRESH = 30000
N_ROWS = 50000
D_IN = 300
NCB = 6
CBW = 64
D_PAD = NCB * CBW
B_NODES = 10000
E_EDGES = 160000
EPT = 10240
E_PAD = 16 * EPT
ECH = 128
NCH = EPT // ECH
B_PAD = 10240
SPT = B_PAD // 16
SCH = SPT // ECH
ACC_ROWS = 16 * 1264
RPT = ACC_ROWS // 16


def _sc_body(xr, nodes_h, esrc_h, edst_h, sums_h, selfo_h,
             nodes_v, src_v, dst_v, slot1d, slotbuf, gb0, acc, sem0):
    c = lax.axis_index("c")
    s = lax.axis_index("s")
    ebase = s * EPT
    sbase = s * SPT
    zrow = s * RPT

    pltpu.sync_copy(nodes_h, nodes_v)
    pltpu.sync_copy(esrc_h.at[pl.ds(ebase, EPT)], src_v)
    pltpu.sync_copy(edst_h.at[pl.ds(ebase, EPT)], dst_v)

    def mk_idx(j, carry):
        for l in range(8):
            off = j * 128 + l * 16
            src16 = src_v[pl.ds(off, 16)]
            dst16 = dst_v[pl.ds(off, 16)]
            nd = jnp.minimum(dst16, B_NODES - 1)
            nval = plsc.load_gather(nodes_v, [nd])
            sa = ((src16 - A_THRESH) >> 31) & 1
            na = ((nval - A_THRESH) >> 31) & 1
            t16 = sa ^ na
            valid = ((dst16 - B_NODES) >> 31) & 1
            slot = valid * (dst16 + t16 * B_NODES - 2 * B_NODES) \
                + 2 * B_NODES
            slot1d[pl.ds(off, 16)] = slot
            src_v[pl.ds(off, 16)] = src16 * NCB
        return carry

    lax.fori_loop(0, NCH, mk_idx, 0)

    def mk_sidx(j, carry):
        for l in range(8):
            off = j * 128 + l * 16
            nv = nodes_v[pl.ds(sbase + off, 16)]
            nodes_v[pl.ds(off, 16)] = nv * NCB
        return carry

    lax.fori_loop(0, SCH, mk_sidx, 0)

    zero16 = jnp.zeros((16,), jnp.float32)

    for p in range(3):
        cb = 3 * c + p

        def mk_gidx(j, carry):
            for l in range(8):
                off = j * 128 + l * 16
                dst_v[pl.ds(off, 16)] = src_v[pl.ds(off, 16)] + cb
            return carry

        lax.fori_loop(0, NCH, mk_gidx, 0)

        def mk_gsidx(j, carry):
            for l in range(8):
                off = j * 128 + l * 16
                nodes_v[pl.ds(SPT + off, 16)] = nodes_v[pl.ds(off, 16)] + cb
            return carry

        lax.fori_loop(0, SCH, mk_gsidx, 0)

        def mk_zero(r, carry):
            for cc in range(CBW // 16):
                gb0[r, pl.ds(cc * 16, 16)] = zero16
            return carry

        lax.fori_loop(0, ECH, mk_zero, 0)

        def do_zero(k, carry):
            pltpu.sync_copy(gb0, acc.at[pl.ds(zrow + k * ECH, ECH)])
            return carry

        lax.fori_loop(0, RPT // ECH, do_zero, 0)
        pltpu.sync_copy(gb0.at[pl.ds(0, RPT % ECH)],
                        acc.at[pl.ds(zrow + (RPT // ECH) * ECH, RPT % ECH)])
        plsc.subcore_barrier()

        def edge_chunk(j, carry):
            for l in range(8):
                slotbuf[0, pl.ds(l * 16, 16)] = \
                    slot1d[pl.ds(j * ECH + l * 16, 16)]
            pltpu.async_copy(xr.at[dst_v.at[pl.ds(j * ECH, ECH)]], gb0,
                             sem0).wait()
            pltpu.sync_copy(gb0, acc.at[slotbuf.at[0]], add=True)
            return carry

        lax.fori_loop(0, NCH, edge_chunk, 0)
        plsc.subcore_barrier()

        def writeout(k, carry):
            pltpu.sync_copy(acc.at[pl.ds(zrow + k * ECH, ECH)],
                            sums_h.at[cb, pl.ds(zrow + k * ECH, ECH)])
            return carry

        lax.fori_loop(0, RPT // ECH, writeout, 0)
        pltpu.sync_copy(
            acc.at[pl.ds(zrow + (RPT // ECH) * ECH, RPT % ECH)],
            sums_h.at[cb, pl.ds(zrow + (RPT // ECH) * ECH, RPT % ECH)])

        def self_chunk(j, carry):
            pltpu.async_copy(
                xr.at[nodes_v.at[pl.ds(SPT + j * ECH, ECH)]], gb0,
                sem0).wait()
            pltpu.sync_copy(gb0, selfo_h.at[cb, pl.ds(sbase + j * ECH, ECH)])
            return carry

        lax.fori_loop(0, SCH, self_chunk, 0)


@functools.cache
def _get_sc_encoder():
    return functools.partial(
        pl.kernel,
        out_type=[
            jax.ShapeDtypeStruct((NCB, ACC_ROWS, CBW), jnp.float32),
            jax.ShapeDtypeStruct((NCB, B_PAD, CBW), jnp.float32),
        ],
        mesh=plsc.VectorSubcoreMesh(core_axis_name="c",
                                    subcore_axis_name="s"),
        compiler_params=pltpu.CompilerParams(needs_layout_passes=False,
                                             use_tc_tiling_on_sc=False),
        scratch_types=[
            pltpu.VMEM((B_PAD,), jnp.int32),
            pltpu.VMEM((EPT,), jnp.int32),
            pltpu.VMEM((EPT,), jnp.int32),
            pltpu.VMEM((EPT,), jnp.int32),
            pltpu.VMEM((1, ECH), jnp.int32),
            pltpu.VMEM((ECH, CBW), jnp.float32),
            pltpu.VMEM_SHARED((ACC_ROWS, CBW), jnp.float32),
            pltpu.SemaphoreType.DMA,
        ],
    )(_sc_body)


def _tc_body(self_ref, s1_ref, s2_ref, w_ref, b_ref, out_ref):
    sf = self_ref[:, :D_IN]

    def feat(ref):
        v = ref[...]
        cnt = v[:, D_IN:D_IN + 1]
        return jnp.where(cnt > 0.0, v[:, :D_IN] / jnp.maximum(cnt, 1.0), 1.0)

    f1 = feat(s1_ref)
    f2 = feat(s2_ref)
    acc = jnp.dot(sf, w_ref[0], preferred_element_type=jnp.float32)
    acc = acc + jnp.dot(f1, w_ref[1], preferred_element_type=jnp.float32)
    acc = acc + jnp.dot(f2, w_ref[2], preferred_element_type=jnp.float32)
    out_ref[...] = acc + b_ref[...]


def _tc_call(selfr, sums, W1r, b1r):
    R = 1000
    grid = B_NODES // R
    return pl.pallas_call(
        _tc_body,
        grid=(grid,),
        in_specs=[
            pl.BlockSpec((R, D_PAD), lambda i: (i, 0)),
            pl.BlockSpec((R, D_PAD), lambda i: (i, 0)),
            pl.BlockSpec((R, D_PAD), lambda i: (i + grid, 0)),
            pl.BlockSpec((3, D_IN, D_IN), lambda i: (0, 0, 0)),
            pl.BlockSpec((1, D_IN), lambda i: (0, 0)),
        ],
        out_specs=pl.BlockSpec((R, D_IN), lambda i: (i, 0)),
        out_shape=jax.ShapeDtypeStruct((B_NODES, D_IN), jnp.float32),
    )(selfr, sums, sums, W1r, b1r)


def kernel(x, W1, b1, nodes, edge_dst, edge_src):
    x = x.astype(jnp.float32)
    nodes = nodes.astype(jnp.int32)
    edge_dst = edge_dst.astype(jnp.int32)
    edge_src = edge_src.astype(jnp.int32)

    xp = jnp.concatenate(
        [x, jnp.ones((N_ROWS, 1), jnp.float32),
         jnp.zeros((N_ROWS, D_PAD - D_IN - 1), jnp.float32)], axis=1)
    xr = xp.reshape(N_ROWS * NCB, CBW)

    nodes_pad = jnp.pad(nodes, (0, B_PAD - B_NODES))
    esrc_pad = jnp.pad(edge_src, (0, E_PAD - E_EDGES))
    edst_pad = jnp.pad(edge_dst, (0, E_PAD - E_EDGES),
                       constant_values=2 * B_NODES)

    sums6, self6 = _get_sc_encoder()(xr, nodes_pad, esrc_pad, edst_pad)
    sums = jnp.transpose(sums6[:, :2 * B_NODES], (1, 0, 2)).reshape(
        2 * B_NODES, D_PAD)
    selfr = jnp.transpose(self6, (1, 0, 2)).reshape(B_PAD, D_PAD)

    W1r = W1.astype(jnp.float32).reshape(3, D_IN, D_IN)
    b1r = b1.astype(jnp.float32).reshape(1, D_IN)
    return _tc_call(selfr, sums, W1r, b1r)
